# baseline (speedup 1.0000x reference)
import functools
import jax
import jax.numpy as jnp
from jax import lax
from jax.experimental import pallas as pl
from jax.experimental.pallas import tpu as pltpu
from jax.experimental.pallas import tpu_sc as plsc

_NO_CAT = 26
_NO_NUM = 13
_NF = _NO_CAT + _NO_NUM
_VOCAB = 100000
_D = 32
_B = 16384

_info = plsc.get_sparse_core_info()
_NC, _NS = _info.num_cores, _info.num_subcores
_NW = _NC * _NS
_BPW = _B // _NW
_CB = 64
_NCHUNK = _BPW // _CB
_NBT = _B // 128

_GSIZES = (8, 6, 6, 6)
_GSTART = (0, 8, 14, 20)
_PAIR_GROUP = (0, 0, 0, 0, 1, 1, 1, 2, 2, 2, 3, 3, 3)

_sc_mesh = plsc.VectorSubcoreMesh(core_axis_name="c", subcore_axis_name="s")


def _select_field(f, raw_v, roff_v, chunk_v):
    zeros = jnp.zeros((16,), dtype=jnp.int32)
    f4 = jnp.full((16,), 4 * f, dtype=jnp.int32)

    def h_body(h, _):
        rows = lax.iota(jnp.int32, 16) + 16 * h
        roff = roff_v[f, pl.ds(16 * h, 16)]
        for d in range(_D):
            v = plsc.load_gather(raw_v, [rows, zeros, roff + d])
            plsc.store_scatter(
                chunk_v,
                [f4 + (d // 8), jnp.full((16,), d % 8, dtype=jnp.int32), rows],
                v,
            )
        return ()

    lax.fori_loop(0, _CB // 16, h_body, ())


@functools.partial(
    pl.kernel,
    mesh=_sc_mesh,
    out_type=jax.ShapeDtypeStruct((4 * _NF, _NBT, 8, 128), jnp.float32),
    scratch_types=[
        pltpu.VMEM((_NO_CAT, _CB), jnp.int32),
        pltpu.VMEM((_NO_CAT, _CB), jnp.int32),
        pltpu.VMEM((4 * _NF, 8, _CB), jnp.float32),
        pltpu.VMEM((_CB, 1, 128), jnp.float32),
        pltpu.VMEM((_CB, 1, 128), jnp.float32),
        pltpu.VMEM((_CB, 1, 128), jnp.float32),
        pltpu.VMEM((_CB, 1, 128), jnp.float32),
        pltpu.SemaphoreType.DMA,
    ],
    compiler_params=pltpu.CompilerParams(
        use_tc_tiling_on_sc=False, needs_layout_passes=False
    ),
)
def _sc_gather(
    g3, roff3, t0, t1, t2, t3, num_hbm, out_hbm,
    gidx_v, roff_v, chunk_v, raw_a, raw_b, raw_c, raw_d, sem,
):
    wid = lax.axis_index("s") * _NC + lax.axis_index("c")
    tabs = (t0, t1, t2, t3)
    bufs = ((raw_a, raw_b), (raw_c, raw_d))

    def fire(p, dsts):
        tab = tabs[_PAIR_GROUP[p]]
        pltpu.async_copy(tab.at[gidx_v.at[2 * p]], dsts[0], sem)
        pltpu.async_copy(tab.at[gidx_v.at[2 * p + 1]], dsts[1], sem)

    def drain2():
        pltpu.make_async_copy(t0.at[gidx_v.at[0]], raw_a, sem).wait()
        pltpu.make_async_copy(t0.at[gidx_v.at[0]], raw_b, sem).wait()

    def chunk(i, _):
        t = wid * _NCHUNK + i
        base = wid * _BPW + i * _CB
        bt = base // 128
        boff = (i % 2) * _CB
        pltpu.sync_copy(g3.at[t], gidx_v)
        pltpu.sync_copy(roff3.at[t], roff_v)
        pltpu.sync_copy(
            num_hbm.at[:, bt, :, pl.ds(boff, _CB)],
            chunk_v.at[pl.ds(4 * _NO_CAT, 4 * _NO_NUM)],
        )
        fire(0, bufs[0])
        for p in range(13):
            if p + 1 < 13:
                fire(p + 1, bufs[(p + 1) % 2])
            drain2()
            _select_field(2 * p, bufs[p % 2][0], roff_v, chunk_v)
            _select_field(2 * p + 1, bufs[p % 2][1], roff_v, chunk_v)
        pltpu.sync_copy(chunk_v, out_hbm.at[:, bt, :, pl.ds(boff, _CB)])
        return ()

    lax.fori_loop(0, _NCHUNK, chunk, ())


def _num_body(x_ref, w_ref, b_ref, o_ref):
    x = x_ref[0]
    w = w_ref[0]
    bb = b_ref[0]
    o_ref[0] = jnp.maximum(
        w[:, None, :, None] * x[None, :, None, :] + bb[:, None, :, None], 0.0
    )


def _num_tc(xr, w4, b4):
    return pl.pallas_call(
        _num_body,
        grid=(_NO_NUM,),
        in_specs=[
            pl.BlockSpec((1, _NBT, 128), lambda j: (j, 0, 0)),
            pl.BlockSpec((1, 4, 8), lambda j: (j, 0, 0)),
            pl.BlockSpec((1, 4, 8), lambda j: (j, 0, 0)),
        ],
        out_specs=pl.BlockSpec((1, 4, _NBT, 8, 128), lambda j: (j, 0, 0, 0, 0)),
        out_shape=jax.ShapeDtypeStruct((_NO_NUM, 4, _NBT, 8, 128), jnp.float32),
    )(xr, w4, b4)


def _repack_body(x_ref, o_ref):
    x3 = x_ref[...].reshape(o_ref.shape[0], 4, _D)
    o_ref[...] = jnp.concatenate([x3[:, k, :] for k in range(4)], axis=1)


def _repack(t2):
    rows = t2.shape[0] // 4
    bs = 5000
    return pl.pallas_call(
        _repack_body,
        grid=(rows // bs,),
        in_specs=[pl.BlockSpec((bs * 4, _D), lambda i: (i, 0))],
        out_specs=pl.BlockSpec((bs, 128), lambda i: (i, 0)),
        out_shape=jax.ShapeDtypeStruct((rows, 128), jnp.float32),
    )(t2)


def kernel(x, tables, W, b):
    idx = x[:, :_NO_CAT].astype(jnp.int32)
    base_f = jnp.repeat(
        jnp.array(_GSTART, dtype=jnp.int32),
        jnp.array(_GSIZES),
        total_repeat_length=_NO_CAT,
    )
    rel = jnp.arange(_NO_CAT, dtype=jnp.int32) - base_f
    flat = idx + rel[None, :] * _VOCAB
    g = flat >> 2
    roff = (flat & 3) * _D
    def arrange(a):
        return a.reshape(_NW * _NCHUNK, _CB, _NO_CAT).transpose(0, 2, 1)

    tabs = []
    for s, n in zip(_GSTART, _GSIZES):
        t2 = tables[s:s + n].reshape(n * _VOCAB, _D)
        tabs.append(_repack(t2).reshape(n * _VOCAB // 4, 1, 128))
    xr = x[:, _NO_CAT:].T.reshape(_NO_NUM, _NBT, 128)
    num5 = _num_tc(
        xr,
        W[:, 0, :].reshape(_NO_NUM, 4, 8),
        b.reshape(_NO_NUM, 4, 8),
    )
    out5 = _sc_gather(
        arrange(g), arrange(roff), *tabs,
        num5.reshape(4 * _NO_NUM, _NBT, 8, 128),
    )
    out = out5.reshape(_NF, 4, _NBT, 8, 128)
    return out.transpose(2, 4, 0, 1, 3).reshape(_B, _NF, _D)


# speedup vs baseline: 1.3913x; 1.3913x over previous
import functools
import jax
import jax.numpy as jnp
from jax import lax
from jax.experimental import pallas as pl
from jax.experimental.pallas import tpu as pltpu
from jax.experimental.pallas import tpu_sc as plsc

_NO_CAT = 26
_NO_NUM = 13
_NF = _NO_CAT + _NO_NUM
_VOCAB = 100000
_D = 32
_B = 16384

_info = plsc.get_sparse_core_info()
_NC, _NS = _info.num_cores, _info.num_subcores
_NW = _NC * _NS
_BPW = _B // _NW
_CB = 64
_NCHUNK = _BPW // _CB
_NBT = _B // 128

_GSIZES = (26,)
_GSTART = (0,)
_PAIR_GROUP = (0,) * 13

_sc_mesh = plsc.VectorSubcoreMesh(core_axis_name="c", subcore_axis_name="s")


def _select_field(f, raw_v, roff_v, chunk_v):
    zeros = jnp.zeros((16,), dtype=jnp.int32)
    f4 = jnp.full((16,), 4 * f, dtype=jnp.int32)

    def h_body(h, _):
        rows = lax.iota(jnp.int32, 16) + 16 * h
        roff = roff_v[f, pl.ds(16 * h, 16)]
        for d in range(_D):
            v = plsc.load_gather(raw_v, [rows, zeros, roff + d])
            plsc.store_scatter(
                chunk_v,
                [f4 + (d // 8), jnp.full((16,), d % 8, dtype=jnp.int32), rows],
                v,
            )
        return ()

    lax.fori_loop(0, _CB // 16, h_body, ())


@functools.partial(
    pl.kernel,
    mesh=_sc_mesh,
    out_type=jax.ShapeDtypeStruct((4 * _NF, _NBT, 8, 128), jnp.float32),
    scratch_types=[
        pltpu.VMEM((_NO_CAT, _CB), jnp.int32),
        pltpu.VMEM((_NO_CAT, _CB), jnp.int32),
        pltpu.VMEM((4 * _NF, 8, _CB), jnp.float32),
        pltpu.VMEM((_CB, 1, 128), jnp.float32),
        pltpu.VMEM((_CB, 1, 128), jnp.float32),
        pltpu.VMEM((_CB, 1, 128), jnp.float32),
        pltpu.VMEM((_CB, 1, 128), jnp.float32),
        pltpu.SemaphoreType.DMA,
    ],
    compiler_params=pltpu.CompilerParams(
        use_tc_tiling_on_sc=False, needs_layout_passes=False
    ),
)
def _sc_gather(
    g3, roff3, t0, num_hbm, out_hbm,
    gidx_v, roff_v, chunk_v, raw_a, raw_b, raw_c, raw_d, sem,
):
    wid = lax.axis_index("s") * _NC + lax.axis_index("c")
    tabs = (t0,)
    bufs = ((raw_a, raw_b), (raw_c, raw_d))

    def fire(p, dsts):
        tab = tabs[_PAIR_GROUP[p]]
        pltpu.async_copy(tab.at[gidx_v.at[2 * p]], dsts[0], sem)
        pltpu.async_copy(tab.at[gidx_v.at[2 * p + 1]], dsts[1], sem)

    def drain2():
        pltpu.make_async_copy(t0.at[gidx_v.at[0]], raw_a, sem).wait()
        pltpu.make_async_copy(t0.at[gidx_v.at[0]], raw_b, sem).wait()

    def chunk(i, _):
        t = wid * _NCHUNK + i
        base = wid * _BPW + i * _CB
        bt = base // 128
        boff = (i % 2) * _CB
        pltpu.sync_copy(g3.at[t], gidx_v)
        pltpu.sync_copy(roff3.at[t], roff_v)
        pltpu.sync_copy(
            num_hbm.at[:, bt, :, pl.ds(boff, _CB)],
            chunk_v.at[pl.ds(4 * _NO_CAT, 4 * _NO_NUM)],
        )
        fire(0, bufs[0])
        for p in range(13):
            if p + 1 < 13:
                fire(p + 1, bufs[(p + 1) % 2])
            drain2()
            _select_field(2 * p, bufs[p % 2][0], roff_v, chunk_v)
            _select_field(2 * p + 1, bufs[p % 2][1], roff_v, chunk_v)
        pltpu.sync_copy(chunk_v, out_hbm.at[:, bt, :, pl.ds(boff, _CB)])
        return ()

    lax.fori_loop(0, _NCHUNK, chunk, ())


def _num_body(x_ref, w_ref, b_ref, o_ref):
    x = x_ref[0]
    w = w_ref[0]
    bb = b_ref[0]
    o_ref[0] = jnp.maximum(
        w[:, None, :, None] * x[None, :, None, :] + bb[:, None, :, None], 0.0
    )


def _num_tc(xr, w4, b4):
    return pl.pallas_call(
        _num_body,
        grid=(_NO_NUM,),
        in_specs=[
            pl.BlockSpec((1, _NBT, 128), lambda j: (j, 0, 0)),
            pl.BlockSpec((1, 4, 8), lambda j: (j, 0, 0)),
            pl.BlockSpec((1, 4, 8), lambda j: (j, 0, 0)),
        ],
        out_specs=pl.BlockSpec((1, 4, _NBT, 8, 128), lambda j: (j, 0, 0, 0, 0)),
        out_shape=jax.ShapeDtypeStruct((_NO_NUM, 4, _NBT, 8, 128), jnp.float32),
    )(xr, w4, b4)


def _repack_body(x_ref, o_ref):
    x3 = x_ref[...].reshape(o_ref.shape[0], 4, _D)
    o_ref[...] = jnp.concatenate([x3[:, k, :] for k in range(4)], axis=1)


def _repack(t2):
    rows = t2.shape[0] // 4
    bs = 5000
    return pl.pallas_call(
        _repack_body,
        grid=(rows // bs,),
        in_specs=[pl.BlockSpec((bs * 4, _D), lambda i: (i, 0))],
        out_specs=pl.BlockSpec((bs, 128), lambda i: (i, 0)),
        out_shape=jax.ShapeDtypeStruct((rows, 128), jnp.float32),
    )(t2)


def kernel(x, tables, W, b):
    idx = x[:, :_NO_CAT].astype(jnp.int32)
    base_f = jnp.repeat(
        jnp.array(_GSTART, dtype=jnp.int32),
        jnp.array(_GSIZES),
        total_repeat_length=_NO_CAT,
    )
    rel = jnp.arange(_NO_CAT, dtype=jnp.int32) - base_f
    flat = idx + rel[None, :] * _VOCAB
    g = flat >> 2
    roff = (flat & 3) * _D
    def arrange(a):
        return a.reshape(_NW * _NCHUNK, _CB, _NO_CAT).transpose(0, 2, 1)

    tabs = []
    for s, n in zip(_GSTART, _GSIZES):
        t2 = tables[s:s + n].reshape(n * _VOCAB, _D)
        tabs.append(_repack(t2).reshape(n * _VOCAB // 4, 1, 128))
    xr = x[:, _NO_CAT:].T.reshape(_NO_NUM, _NBT, 128)
    num5 = _num_tc(
        xr,
        W[:, 0, :].reshape(_NO_NUM, 4, 8),
        b.reshape(_NO_NUM, 4, 8),
    )
    out5 = _sc_gather(
        arrange(g), arrange(roff), *tabs,
        num5.reshape(4 * _NO_NUM, _NBT, 8, 128),
    )
    out = out5.reshape(_NF, 4, _NBT, 8, 128)
    return out.transpose(2, 4, 0, 1, 3).reshape(_B, _NF, _D)
